# Initial kernel scaffold; baseline (speedup 1.0000x reference)
import jax
import jax.numpy as jnp
from jax.experimental import pallas as pl


def kernel(q, k, v, mask, mem_keys, mem_vals, scale_param):
    raise NotImplementedError("write your pallas kernel here")


# speedup vs baseline: 9.3227x; 9.3227x over previous
import jax
import jax.numpy as jnp
from jax import lax
from jax.experimental import pallas as pl
from jax.experimental.pallas import tpu as pltpu

KNN = 32
NEG = -1e30


def _l2n(x):
    ss = jnp.sum(x * x, axis=-1, keepdims=True)
    return x / jnp.maximum(jnp.sqrt(ss), 1e-12)


def _attn_body(scale_ref, q_ref, k_ref, v_ref, maskf_ref, mk_ref, mv_ref,
               o_ref):
    h = pl.program_id(0)
    scale = jnp.exp(jnp.full((1, 1), scale_ref[h], jnp.float32))

    qn = _l2n(q_ref[0, 0])
    kn = _l2n(k_ref[0])
    mkn = _l2n(mk_ref[0])

    smem = lax.dot_general(qn, mkn, (((1,), (1,)), ((), ())),
                           preferred_element_type=jnp.float32)

    def step(i, w):
        m = jnp.max(w, axis=-1, keepdims=True)
        return jnp.where(w == m, NEG, w)

    w = lax.fori_loop(0, KNN, step, smem)
    lm = jnp.where(w == NEG, smem * scale, NEG)

    sl = lax.dot_general(qn, kn, (((1,), (1,)), ((), ())),
                         preferred_element_type=jnp.float32) * scale
    sl = sl + NEG * (1.0 - maskf_ref[0])[None, :]

    mx = jnp.maximum(jnp.max(lm, axis=-1, keepdims=True),
                     jnp.max(sl, axis=-1, keepdims=True))
    pm = jnp.exp(lm - mx)
    pll = jnp.exp(sl - mx)
    z = (jnp.sum(pm, axis=-1, keepdims=True) +
         jnp.sum(pll, axis=-1, keepdims=True))
    out = (jnp.dot(pm, mv_ref[0], preferred_element_type=jnp.float32) +
           jnp.dot(pll, v_ref[0], preferred_element_type=jnp.float32)) / z
    o_ref[0, 0] = out


@jax.jit
def kernel(q, k, v, mask, mem_keys, mem_vals, scale_param):
    B, H, S, D = q.shape
    M = mem_keys.shape[1]
    BQ = 128 if S % 128 == 0 else S
    maskf = mask.astype(jnp.float32)
    scales = scale_param.reshape(H)

    return pl.pallas_call(
        _attn_body,
        grid=(H, S // BQ),
        in_specs=[
            pl.BlockSpec((H,), lambda h, i: (0,), memory_space=pltpu.SMEM),
            pl.BlockSpec((1, 1, BQ, D), lambda h, i: (0, h, i, 0)),
            pl.BlockSpec((1, S, D), lambda h, i: (0, 0, 0)),
            pl.BlockSpec((1, S, D), lambda h, i: (0, 0, 0)),
            pl.BlockSpec((1, S), lambda h, i: (0, 0)),
            pl.BlockSpec((1, M, D), lambda h, i: (0, 0, 0)),
            pl.BlockSpec((1, M, D), lambda h, i: (0, 0, 0)),
        ],
        out_specs=pl.BlockSpec((1, 1, BQ, D), lambda h, i: (0, h, i, 0)),
        out_shape=jax.ShapeDtypeStruct((B, H, S, D), jnp.float32),
    )(scales, q, k, v, maskf, mem_keys, mem_vals)


# speedup vs baseline: 24.2989x; 2.6064x over previous
import jax
import jax.numpy as jnp
from jax import lax
from jax.experimental import pallas as pl
from jax.experimental.pallas import tpu as pltpu

KNN = 32
NEG = -1e30


def _l2n(x):
    ss = jnp.sum(x * x, axis=-1, keepdims=True)
    return x / jnp.maximum(jnp.sqrt(ss), 1e-12)


def _keyify(x):
    s = lax.bitcast_convert_type(x, jnp.int32)
    return s ^ (lax.shift_right_arithmetic(s, 31) & jnp.int32(0x7FFFFFFF))


def _attn_body(scale_ref, q_ref, k_ref, v_ref, maskf_ref, mk_ref, mv_ref,
               o_ref, kn_ref, mkn_ref):
    h = pl.program_id(0)
    qi = pl.program_id(1)

    @pl.when(jnp.logical_and(h == 0, qi == 0))
    def _():
        kn_ref[...] = _l2n(k_ref[0])
        mkn_ref[...] = _l2n(mk_ref[0])

    scale = jnp.exp(jnp.full((1, 1), scale_ref[h], jnp.float32))
    qn = _l2n(q_ref[0, 0])

    smem = lax.dot_general(qn, mkn_ref[...], (((1,), (1,)), ((), ())),
                           preferred_element_type=jnp.float32)
    bq, m = smem.shape

    keys = _keyify(smem)
    cm = jnp.max(smem.reshape(bq, KNN, m // KNN), axis=-1)
    t_lo = jnp.min(cm, axis=-1, keepdims=True)
    t_hi = jnp.max(cm, axis=-1, keepdims=True)
    il0 = _keyify(t_lo)
    ih0 = _keyify(t_hi) + 1

    def cond(st):
        il, ih, clo = st
        return jnp.any(jnp.logical_and(ih - il > 1, clo != KNN))

    def body(st):
        il, ih, clo = st
        mid = il + lax.shift_right_arithmetic(ih - il, 1)
        c = jnp.sum((keys >= mid).astype(jnp.int32), axis=-1, keepdims=True)
        ge = c >= KNN
        return (jnp.where(ge, mid, il), jnp.where(ge, ih, mid),
                jnp.where(ge, c, clo))

    il, _, _ = lax.while_loop(cond, body, (il0, ih0, jnp.full_like(il0, m)))
    lm = jnp.where(keys >= il, smem * scale, NEG)

    sl = lax.dot_general(qn, kn_ref[...], (((1,), (1,)), ((), ())),
                         preferred_element_type=jnp.float32) * scale
    sl = sl + NEG * (1.0 - maskf_ref[0])[None, :]

    mx = jnp.maximum(jnp.max(lm, axis=-1, keepdims=True),
                     jnp.max(sl, axis=-1, keepdims=True))
    pm = jnp.exp(lm - mx)
    pll = jnp.exp(sl - mx)
    z = (jnp.sum(pm, axis=-1, keepdims=True) +
         jnp.sum(pll, axis=-1, keepdims=True))
    out = (jnp.dot(pm, mv_ref[0], preferred_element_type=jnp.float32) +
           jnp.dot(pll, v_ref[0], preferred_element_type=jnp.float32)) / z
    o_ref[0, 0] = out


@jax.jit
def kernel(q, k, v, mask, mem_keys, mem_vals, scale_param):
    B, H, S, D = q.shape
    M = mem_keys.shape[1]
    BQ = 128 if S % 128 == 0 else S
    maskf = mask.astype(jnp.float32)
    scales = scale_param.reshape(H)

    return pl.pallas_call(
        _attn_body,
        grid=(H, S // BQ),
        in_specs=[
            pl.BlockSpec((H,), lambda h, i: (0,), memory_space=pltpu.SMEM),
            pl.BlockSpec((1, 1, BQ, D), lambda h, i: (0, h, i, 0)),
            pl.BlockSpec((1, S, D), lambda h, i: (0, 0, 0)),
            pl.BlockSpec((1, S, D), lambda h, i: (0, 0, 0)),
            pl.BlockSpec((1, S), lambda h, i: (0, 0)),
            pl.BlockSpec((1, M, D), lambda h, i: (0, 0, 0)),
            pl.BlockSpec((1, M, D), lambda h, i: (0, 0, 0)),
        ],
        out_specs=pl.BlockSpec((1, 1, BQ, D), lambda h, i: (0, h, i, 0)),
        out_shape=jax.ShapeDtypeStruct((B, H, S, D), jnp.float32),
        scratch_shapes=[
            pltpu.VMEM((S, D), jnp.float32),
            pltpu.VMEM((M, D), jnp.float32),
        ],
    )(scales, q, k, v, maskf, mem_keys, mem_vals)
